# baseline (speedup 1.0000x reference)
import math

import jax
import jax.numpy as jnp
from jax.experimental import pallas as pl
from jax.experimental.pallas import tpu as pltpu

N = 2048
H = 16
BI = 128
NJ = N // 128
NBLK = 2 * (NJ - 1) + 3
K = 8
STEPS = H * NJ


def _body(tbl_ref, out_ref, strip_ref, sems):
    h = pl.program_id(0)
    i = pl.program_id(1)

    @pl.when(i == 0)
    def _build_master_strip():
        p = h % 2
        c_lo = tbl_ref[15, h]
        c_hi = tbl_ref[31, h]
        strip_ref[p, :, 0:NJ - 1, :] = jnp.full((BI, NJ - 1, 128), c_lo, jnp.float32)
        strip_ref[p, :, NJ + 2:NBLK, :] = jnp.full((BI, NJ - 1, 128), c_hi, jnp.float32)
        r = jax.lax.broadcasted_iota(jnp.int32, (BI, 3, 128), 0)
        t = jax.lax.broadcasted_iota(jnp.int32, (BI, 3, 128), 1)
        c = jax.lax.broadcasted_iota(jnp.int32, (BI, 3, 128), 2)
        rel = (c + (t - 1) * 128) - r
        nn = -rel
        ret = (nn < 0).astype(jnp.int32) * 16
        na = jnp.abs(nn)
        is_small = na < 8
        vl = 8 + (
            jnp.log(na.astype(jnp.float32) / 8.0) / math.log(16.0) * 8.0
        ).astype(jnp.int32)
        vl = jnp.minimum(vl, jnp.full_like(vl, 15))
        bucket = ret + jnp.where(is_small, na, vl)
        acc = jnp.zeros((BI, 3, 128), jnp.float32)
        for k in range(32):
            acc = acc + jnp.where(bucket == k, tbl_ref[k, h], 0.0)
        strip_ref[p, :, NJ - 1:NJ + 2, :] = acc

    def copy_for(ii):
        return pltpu.make_async_copy(
            strip_ref.at[h % 2, :, pl.ds(NJ - ii, NJ), :],
            out_ref.at[h, pl.ds(ii * BI, BI), :, :],
            sems.at[ii % K],
        )

    @pl.when(i >= K)
    def _wait_oldest():
        copy_for(i - K).wait()

    copy_for(i).start()

    @pl.when(i == NJ - 1)
    def _drain():
        for d in range(K):
            copy_for(NJ - K + d).wait()


def kernel(n, rel_bias_table):
    del n
    out = pl.pallas_call(
        _body,
        grid=(H, NJ),
        in_specs=[pl.BlockSpec(memory_space=pltpu.SMEM)],
        out_specs=pl.BlockSpec(memory_space=pl.ANY),
        out_shape=jax.ShapeDtypeStruct((H, N, NJ, 128), jnp.float32),
        scratch_shapes=[
            pltpu.VMEM((2, BI, NBLK, 128), jnp.float32),
            pltpu.SemaphoreType.DMA((K,)),
        ],
        compiler_params=pltpu.CompilerParams(
            dimension_semantics=("parallel", "arbitrary"),
        ),
    )(rel_bias_table)
    return out.reshape(H, N, N)


# speedup vs baseline: 1.1149x; 1.1149x over previous
import math

import jax
import jax.numpy as jnp
from jax.experimental import pallas as pl
from jax.experimental.pallas import tpu as pltpu

N = 2048
H = 16
BI = 128
NJ = N // 128
NBLK = 2 * (NJ - 1) + 3
K = 16
STEPS = H * NJ


def _body(tbl_ref, out_ref, strip_ref, sems):
    h = pl.program_id(0)
    i = pl.program_id(1)

    @pl.when(i == 0)
    def _build_master_strip():
        p = h % 2
        c_lo = tbl_ref[15, h]
        c_hi = tbl_ref[31, h]
        strip_ref[p, :, 0:NJ - 1, :] = jnp.full((BI, NJ - 1, 128), c_lo, jnp.float32)
        strip_ref[p, :, NJ + 2:NBLK, :] = jnp.full((BI, NJ - 1, 128), c_hi, jnp.float32)
        r = jax.lax.broadcasted_iota(jnp.int32, (BI, 3, 128), 0)
        t = jax.lax.broadcasted_iota(jnp.int32, (BI, 3, 128), 1)
        c = jax.lax.broadcasted_iota(jnp.int32, (BI, 3, 128), 2)
        rel = (c + (t - 1) * 128) - r
        nn = -rel
        ret = (nn < 0).astype(jnp.int32) * 16
        na = jnp.abs(nn)
        is_small = na < 8
        vl = 8 + (
            jnp.log(na.astype(jnp.float32) / 8.0) / math.log(16.0) * 8.0
        ).astype(jnp.int32)
        vl = jnp.minimum(vl, jnp.full_like(vl, 15))
        bucket = ret + jnp.where(is_small, na, vl)
        acc = jnp.zeros((BI, 3, 128), jnp.float32)
        for k in range(32):
            acc = acc + jnp.where(bucket == k, tbl_ref[k, h], 0.0)
        strip_ref[p, :, NJ - 1:NJ + 2, :] = acc

    s = h * NJ + i

    def copy_for(step):
        hh = step // NJ
        ii = step % NJ
        return pltpu.make_async_copy(
            strip_ref.at[hh % 2, :, pl.ds(NJ - ii, NJ), :],
            out_ref.at[hh, pl.ds(ii * BI, BI), :, :],
            sems.at[step % K],
        )

    @pl.when(s >= K)
    def _wait_oldest():
        copy_for(s - K).wait()

    copy_for(s).start()

    @pl.when(s == STEPS - 1)
    def _drain():
        for d in range(K):
            copy_for(STEPS - K + d).wait()


def kernel(n, rel_bias_table):
    del n
    out = pl.pallas_call(
        _body,
        grid=(H, NJ),
        in_specs=[pl.BlockSpec(memory_space=pltpu.SMEM)],
        out_specs=pl.BlockSpec(memory_space=pl.ANY),
        out_shape=jax.ShapeDtypeStruct((H, N, NJ, 128), jnp.float32),
        scratch_shapes=[
            pltpu.VMEM((2, BI, NBLK, 128), jnp.float32),
            pltpu.SemaphoreType.DMA((K,)),
        ],
        compiler_params=pltpu.CompilerParams(
            dimension_semantics=("arbitrary", "arbitrary"),
        ),
    )(rel_bias_table)
    return out.reshape(H, N, N)


# speedup vs baseline: 1.4420x; 1.2934x over previous
import math

import jax
import jax.numpy as jnp
from jax import lax
from jax.experimental import pallas as pl
from jax.experimental.pallas import tpu as pltpu
from jax.experimental.pallas import tpu_sc as plsc

N = 2048
H = 16
NPH = 16
WX = 3968
NC = 2
NS = 16
UPW = H * NPH // (NC * NS)


def _wbuild_body(tbl_ref, w_ref):
    h = pl.program_id(0)
    m = pl.program_id(1)
    r = lax.broadcasted_iota(jnp.int32, (8, WX), 0)
    x = lax.broadcasted_iota(jnp.int32, (8, WX), 1)
    rel = (x + 8 * m + 7 - r) - (N - 1)
    nn = -rel
    ret = (nn < 0).astype(jnp.int32) * 16
    na = jnp.abs(nn)
    is_small = na < 8
    vl = 8 + (
        jnp.log(na.astype(jnp.float32) / 8.0) / math.log(16.0) * 8.0
    ).astype(jnp.int32)
    vl = jnp.minimum(vl, jnp.full_like(vl, 15))
    bucket = ret + jnp.where(is_small, na, vl)
    acc = jnp.zeros((8, WX), jnp.float32)
    for k in range(32):
        acc = acc + jnp.where(bucket == k, tbl_ref[k, h], 0.0)
    w_ref[0, 0] = acc


def _sc_body(w_hbm, out_hbm, w_v, sems):
    wid = lax.axis_index("s") * NC + lax.axis_index("c")

    def unit_writes(u, do):
        g = wid * UPW + u
        h = g // NPH
        m = g % NPH
        b = u % 2
        for p in range(NPH):
            i8 = pl.multiple_of((N - 8) - 8 * m - 128 * p, 8)
            cp = pltpu.make_async_copy(
                w_v.at[b, :, pl.ds(128 * p, N)],
                out_hbm.at[h, pl.ds(i8, 8), :],
                sems.at[b],
            )
            if do == "start":
                cp.start()
            else:
                cp.wait()

    for u in range(UPW):
        g = wid * UPW + u
        pltpu.sync_copy(w_hbm.at[g // NPH, g % NPH], w_v.at[u % 2])
        unit_writes(u, "start")
        if u >= 1:
            unit_writes(u - 1, "wait")
    unit_writes(UPW - 1, "wait")


def kernel(n, rel_bias_table):
    del n
    w = pl.pallas_call(
        _wbuild_body,
        grid=(H, NPH),
        in_specs=[pl.BlockSpec(memory_space=pltpu.SMEM)],
        out_specs=pl.BlockSpec((1, 1, 8, WX), lambda h, m: (h, m, 0, 0)),
        out_shape=jax.ShapeDtypeStruct((H, NPH, 8, WX), jnp.float32),
    )(rel_bias_table)
    expand = pl.kernel(
        _sc_body,
        out_type=jax.ShapeDtypeStruct((H, N, N), jnp.float32),
        mesh=plsc.VectorSubcoreMesh(
            core_axis_name="c", subcore_axis_name="s",
            num_cores=NC, num_subcores=NS,
        ),
        scratch_types=[
            pltpu.VMEM((2, 8, WX), jnp.float32),
            pltpu.SemaphoreType.DMA((2,)),
        ],
    )
    return expand(w)


# speedup vs baseline: 1.5246x; 1.0573x over previous
import math

import jax
import jax.numpy as jnp
from jax import lax
from jax.experimental import pallas as pl
from jax.experimental.pallas import tpu as pltpu
from jax.experimental.pallas import tpu_sc as plsc

N = 2048
H = 16
NPH = 16
WX = 3968
NC = 2
NS = 16
UPW = H * NPH // (NC * NS)


BX0 = 1792
BW = 512
BSW = 640


def _band_body(tbl_ref, band_ref):
    h = pl.program_id(0)
    r = lax.broadcasted_iota(jnp.int32, (8, BSW), 0)
    z = lax.broadcasted_iota(jnp.int32, (8, BSW), 1)
    rel = (z + BX0 + 7 - r) - (N - 1)
    nn = -rel
    ret = (nn < 0).astype(jnp.int32) * 16
    na = jnp.abs(nn)
    is_small = na < 8
    vl = 8 + (
        jnp.log(na.astype(jnp.float32) / 8.0) / math.log(16.0) * 8.0
    ).astype(jnp.int32)
    vl = jnp.minimum(vl, jnp.full_like(vl, 15))
    bucket = ret + jnp.where(is_small, na, vl)
    acc = jnp.zeros((8, BSW), jnp.float32)
    for k in range(32):
        acc = acc + jnp.where(bucket == k, tbl_ref[k, h], 0.0)
    for m in range(NPH):
        band_ref[0, m] = acc[:, 8 * m:8 * m + BW]


def _wbuild_body(tbl_ref, band_ref, w_ref):
    h = pl.program_id(0)
    m = pl.program_id(1)
    c_lo = tbl_ref[15, h]
    c_hi = tbl_ref[31, h]
    r = lax.broadcasted_iota(jnp.int32, (8, WX), 0)
    x = lax.broadcasted_iota(jnp.int32, (8, WX), 1)
    rel = (x + 8 * m + 7 - r) - (N - 1)
    w_ref[0, 0] = jnp.where(rel >= 0, c_hi, c_lo)
    w_ref[0, 0, :, BX0:BX0 + BW] = band_ref[0, 0]


def _sc_body(w_hbm, out_hbm, w_v, sems):
    wid = lax.axis_index("s") * NC + lax.axis_index("c")

    def unit_writes(u, do):
        g = wid * UPW + u
        h = g // NPH
        m = g % NPH
        b = u % 2
        for p in range(NPH):
            i8 = pl.multiple_of((N - 8) - 8 * m - 128 * p, 8)
            cp = pltpu.make_async_copy(
                w_v.at[b, :, pl.ds(128 * p, N)],
                out_hbm.at[h, pl.ds(i8, 8), :],
                sems.at[b],
            )
            if do == "start":
                cp.start()
            else:
                cp.wait()

    for u in range(UPW):
        g = wid * UPW + u
        pltpu.sync_copy(w_hbm.at[g // NPH, g % NPH], w_v.at[u % 2])
        unit_writes(u, "start")
        if u >= 1:
            unit_writes(u - 1, "wait")
    unit_writes(UPW - 1, "wait")


def kernel(n, rel_bias_table):
    del n
    band = pl.pallas_call(
        _band_body,
        grid=(H,),
        in_specs=[pl.BlockSpec(memory_space=pltpu.SMEM)],
        out_specs=pl.BlockSpec((1, NPH, 8, BW), lambda h: (h, 0, 0, 0)),
        out_shape=jax.ShapeDtypeStruct((H, NPH, 8, BW), jnp.float32),
    )(rel_bias_table)
    w = pl.pallas_call(
        _wbuild_body,
        grid=(H, NPH),
        in_specs=[
            pl.BlockSpec(memory_space=pltpu.SMEM),
            pl.BlockSpec((1, 1, 8, BW), lambda h, m: (h, m, 0, 0)),
        ],
        out_specs=pl.BlockSpec((1, 1, 8, WX), lambda h, m: (h, m, 0, 0)),
        out_shape=jax.ShapeDtypeStruct((H, NPH, 8, WX), jnp.float32),
    )(rel_bias_table, band)
    expand = pl.kernel(
        _sc_body,
        out_type=jax.ShapeDtypeStruct((H, N, N), jnp.float32),
        mesh=plsc.VectorSubcoreMesh(
            core_axis_name="c", subcore_axis_name="s",
            num_cores=NC, num_subcores=NS,
        ),
        scratch_types=[
            pltpu.VMEM((2, 8, WX), jnp.float32),
            pltpu.SemaphoreType.DMA((2,)),
        ],
    )
    return expand(w)


# speedup vs baseline: 3.3251x; 2.1810x over previous
import math

import jax
import jax.numpy as jnp
from jax import lax
from jax.experimental import pallas as pl
from jax.experimental.pallas import tpu as pltpu
from jax.experimental.pallas import tpu_sc as plsc

N = 2048
H = 16
NPH = 16
WX = 3968
BX0 = 1792
BW = 512
BSW = 640
NC = 2
NS = 16


def _band_body(tbl_ref, band_ref, fill_ref):
    h = pl.program_id(0)
    c_lo = tbl_ref[15, h]
    c_hi = tbl_ref[31, h]
    fill_ref[0, :, 0:BX0] = jnp.full((8, BX0), c_lo, jnp.float32)
    fill_ref[0, :, BX0:WX] = jnp.full((8, WX - BX0), c_hi, jnp.float32)
    r = lax.broadcasted_iota(jnp.int32, (8, BSW), 0)
    z = lax.broadcasted_iota(jnp.int32, (8, BSW), 1)
    rel = (z + BX0 + 7 - r) - (N - 1)
    nn = -rel
    ret = (nn < 0).astype(jnp.int32) * 16
    na = jnp.abs(nn)
    is_small = na < 8
    vl = 8 + (
        jnp.log(na.astype(jnp.float32) / 8.0) / math.log(16.0) * 8.0
    ).astype(jnp.int32)
    vl = jnp.minimum(vl, jnp.full_like(vl, 15))
    bucket = ret + jnp.where(is_small, na, vl)
    acc = jnp.zeros((8, BSW), jnp.float32)
    for k in range(32):
        acc = acc + jnp.where(bucket == k, tbl_ref[k, h], 0.0)
    for m in range(NPH):
        band_ref[0, m] = acc[:, 8 * m:8 * m + BW]


def _pieces(p):
    w_left = BX0 - 128 * p
    yb = max(0, BX0 - 128 * p)
    bw = min(BX0 + BW, 128 * p + N) - max(BX0, 128 * p)
    soff = max(BX0, 128 * p) - BX0
    w_right = N - (yb + bw)
    out = []
    if w_left > 0:
        out.append((0, w_left, "fill", 0))
    out.append((yb, bw, "band", soff))
    if w_right > 0:
        out.append((yb + bw, w_right, "fill", BX0 + BW))
    return out


def _sc_body(band_hbm, fill_hbm, out_hbm, fill_v, band_v, sems):
    wid = lax.axis_index("s") * NC + lax.axis_index("c")
    h = wid // 2
    mhalf = (wid % 2) * 8
    pltpu.sync_copy(fill_hbm.at[h], fill_v)

    def unit_writes(u, do):
        m = mhalf + u
        b = u % 2
        for p in range(NPH):
            i8 = pl.multiple_of((N - 8) - 8 * m - 128 * p, 8)
            for (y0, w, src, soff) in _pieces(p):
                ref = (fill_v.at[:, pl.ds(soff, w)] if src == "fill"
                       else band_v.at[b, :, pl.ds(soff, w)])
                cp = pltpu.make_async_copy(
                    ref, out_hbm.at[h, pl.ds(i8, 8), pl.ds(y0, w)], sems.at[b])
                if do == "start":
                    cp.start()
                else:
                    cp.wait()

    for u in range(8):
        if u >= 2:
            unit_writes(u - 2, "wait")
        pltpu.sync_copy(band_hbm.at[h, mhalf + u], band_v.at[u % 2])
        unit_writes(u, "start")
    unit_writes(6, "wait")
    unit_writes(7, "wait")


def kernel(n, rel_bias_table):
    del n
    band, fill = pl.pallas_call(
        _band_body,
        grid=(H,),
        in_specs=[pl.BlockSpec(memory_space=pltpu.SMEM)],
        out_specs=[
            pl.BlockSpec((1, NPH, 8, BW), lambda h: (h, 0, 0, 0)),
            pl.BlockSpec((1, 8, WX), lambda h: (h, 0, 0)),
        ],
        out_shape=[
            jax.ShapeDtypeStruct((H, NPH, 8, BW), jnp.float32),
            jax.ShapeDtypeStruct((H, 8, WX), jnp.float32),
        ],
    )(rel_bias_table)
    expand = pl.kernel(
        _sc_body,
        out_type=jax.ShapeDtypeStruct((H, N, N), jnp.float32),
        mesh=plsc.VectorSubcoreMesh(
            core_axis_name="c", subcore_axis_name="s",
            num_cores=NC, num_subcores=NS,
        ),
        scratch_types=[
            pltpu.VMEM((8, WX), jnp.float32),
            pltpu.VMEM((2, 8, BW), jnp.float32),
            pltpu.SemaphoreType.DMA((2,)),
        ],
    )
    return expand(band, fill)


# speedup vs baseline: 3.3336x; 1.0026x over previous
import math

import jax
import jax.numpy as jnp
from jax import lax
from jax.experimental import pallas as pl
from jax.experimental.pallas import tpu as pltpu
from jax.experimental.pallas import tpu_sc as plsc

N = 2048
H = 16
NPH = 16
WX = 3968
BX0 = 1792
BW = 384
BSW = 640
NC = 2
NS = 16


def _band_body(tbl_ref, band_ref, fill_ref):
    h = pl.program_id(0)
    c_lo = tbl_ref[15, h]
    c_hi = tbl_ref[31, h]
    fill_ref[0, :, 0:BX0] = jnp.full((8, BX0), c_lo, jnp.float32)
    fill_ref[0, :, BX0:WX] = jnp.full((8, WX - BX0), c_hi, jnp.float32)
    r = lax.broadcasted_iota(jnp.int32, (8, BSW), 0)
    z = lax.broadcasted_iota(jnp.int32, (8, BSW), 1)
    rel = (z + BX0 + 7 - r) - (N - 1)
    nn = -rel
    ret = (nn < 0).astype(jnp.int32) * 16
    na = jnp.abs(nn)
    is_small = na < 8
    vl = 8 + (
        jnp.log(na.astype(jnp.float32) / 8.0) / math.log(16.0) * 8.0
    ).astype(jnp.int32)
    vl = jnp.minimum(vl, jnp.full_like(vl, 15))
    bucket = ret + jnp.where(is_small, na, vl)
    acc = jnp.zeros((8, BSW), jnp.float32)
    for k in range(32):
        acc = acc + jnp.where(bucket == k, tbl_ref[k, h], 0.0)
    for m in range(NPH):
        band_ref[0, m] = acc[:, 8 * m:8 * m + BW]


def _pieces(p):
    w_left = BX0 - 128 * p
    yb = max(0, BX0 - 128 * p)
    bw = min(BX0 + BW, 128 * p + N) - max(BX0, 128 * p)
    soff = max(BX0, 128 * p) - BX0
    w_right = N - (yb + bw)
    out = []
    if w_left > 0:
        out.append((0, w_left, "fill", 0))
    if bw > 0:
        out.append((yb, bw, "band", soff))
    if w_right > 0:
        out.append((yb + bw, w_right, "fill", BX0 + BW))
    return out


def _sc_body(band_hbm, fill_hbm, out_hbm, fill_v, band_v, sems):
    wid = lax.axis_index("s") * NC + lax.axis_index("c")
    h = wid // 2
    mhalf = (wid % 2) * 8
    pltpu.sync_copy(fill_hbm.at[h], fill_v)

    def unit_writes(u, do):
        m = mhalf + u
        b = u % 2
        for p in range(NPH):
            i8 = pl.multiple_of((N - 8) - 8 * m - 128 * p, 8)
            for (y0, w, src, soff) in _pieces(p):
                ref = (fill_v.at[:, pl.ds(soff, w)] if src == "fill"
                       else band_v.at[b, :, pl.ds(soff, w)])
                cp = pltpu.make_async_copy(
                    ref, out_hbm.at[h, pl.ds(i8, 8), pl.ds(y0, w)], sems.at[b])
                if do == "start":
                    cp.start()
                else:
                    cp.wait()

    for u in range(8):
        if u >= 2:
            unit_writes(u - 2, "wait")
        pltpu.sync_copy(band_hbm.at[h, mhalf + u], band_v.at[u % 2])
        unit_writes(u, "start")
    unit_writes(6, "wait")
    unit_writes(7, "wait")


def kernel(n, rel_bias_table):
    del n
    band, fill = pl.pallas_call(
        _band_body,
        grid=(H,),
        in_specs=[pl.BlockSpec(memory_space=pltpu.SMEM)],
        out_specs=[
            pl.BlockSpec((1, NPH, 8, BW), lambda h: (h, 0, 0, 0)),
            pl.BlockSpec((1, 8, WX), lambda h: (h, 0, 0)),
        ],
        out_shape=[
            jax.ShapeDtypeStruct((H, NPH, 8, BW), jnp.float32),
            jax.ShapeDtypeStruct((H, 8, WX), jnp.float32),
        ],
    )(rel_bias_table)
    expand = pl.kernel(
        _sc_body,
        out_type=jax.ShapeDtypeStruct((H, N, N), jnp.float32),
        mesh=plsc.VectorSubcoreMesh(
            core_axis_name="c", subcore_axis_name="s",
            num_cores=NC, num_subcores=NS,
        ),
        scratch_types=[
            pltpu.VMEM((8, WX), jnp.float32),
            pltpu.VMEM((2, 8, BW), jnp.float32),
            pltpu.SemaphoreType.DMA((2,)),
        ],
    )
    return expand(band, fill)


# speedup vs baseline: 3.3610x; 1.0082x over previous
import math

import jax
import jax.numpy as jnp
from jax import lax
from jax.experimental import pallas as pl
from jax.experimental.pallas import tpu as pltpu
from jax.experimental.pallas import tpu_sc as plsc

N = 2048
H = 16
NPH = 16
WX = 3968
BX0 = 1792
BW = 384
BSW = 640
NC = 2
NS = 16


def _band_body(tbl_ref, band_ref, fill_ref):
    h = pl.program_id(0)
    c_lo = tbl_ref[15, h]
    c_hi = tbl_ref[31, h]
    fill_ref[0, :, 0:BX0] = jnp.full((8, BX0), c_lo, jnp.float32)
    fill_ref[0, :, BX0:WX] = jnp.full((8, WX - BX0), c_hi, jnp.float32)
    r = lax.broadcasted_iota(jnp.int32, (8, BSW), 0)
    z = lax.broadcasted_iota(jnp.int32, (8, BSW), 1)
    rel = (z + BX0 + 7 - r) - (N - 1)
    nn = -rel
    ret = (nn < 0).astype(jnp.int32) * 16
    na = jnp.abs(nn)
    is_small = na < 8
    vl = 8 + (
        jnp.log(na.astype(jnp.float32) / 8.0) / math.log(16.0) * 8.0
    ).astype(jnp.int32)
    vl = jnp.minimum(vl, jnp.full_like(vl, 15))
    bucket = ret + jnp.where(is_small, na, vl)
    acc = jnp.zeros((8, BSW), jnp.float32)
    for k in range(32):
        acc = acc + jnp.where(bucket == k, tbl_ref[k, h], 0.0)
    for m in range(NPH):
        band_ref[0, m] = acc[:, 8 * m:8 * m + BW]


def _pieces(p):
    w_left = BX0 - 128 * p
    yb = max(0, BX0 - 128 * p)
    bw = min(BX0 + BW, 128 * p + N) - max(BX0, 128 * p)
    soff = max(BX0, 128 * p) - BX0
    w_right = N - (yb + bw)
    out = []
    if w_left > 0:
        out.append((0, w_left, "fill", 0))
    if bw > 0:
        out.append((yb, bw, "band", soff))
    if w_right > 0:
        out.append((yb + bw, w_right, "fill", BX0 + BW))
    return out


def _sc_body(band_hbm, fill_hbm, out_hbm, fill_v, band_v, sems, bsems):
    wid = lax.axis_index("s") * NC + lax.axis_index("c")
    h = wid // 2
    mhalf = (wid % 2) * 8

    def band_load(u):
        return pltpu.make_async_copy(
            band_hbm.at[h, mhalf + u], band_v.at[u % 3], bsems.at[u % 3])

    band_load(0).start()
    pltpu.sync_copy(fill_hbm.at[h], fill_v)

    def unit_writes(u, do):
        m = mhalf + u
        b = u % 3
        for p in range(NPH):
            i8 = pl.multiple_of((N - 8) - 8 * m - 128 * p, 8)
            for (y0, w, src, soff) in _pieces(p):
                ref = (fill_v.at[:, pl.ds(soff, w)] if src == "fill"
                       else band_v.at[b, :, pl.ds(soff, w)])
                cp = pltpu.make_async_copy(
                    ref, out_hbm.at[h, pl.ds(i8, 8), pl.ds(y0, w)], sems.at[b])
                if do == "start":
                    cp.start()
                else:
                    cp.wait()

    for u in range(8):
        if u >= 2:
            unit_writes(u - 2, "wait")
        if u < 7:
            band_load(u + 1).start()
        band_load(u).wait()
        unit_writes(u, "start")
    unit_writes(6, "wait")
    unit_writes(7, "wait")


def kernel(n, rel_bias_table):
    del n
    band, fill = pl.pallas_call(
        _band_body,
        grid=(H,),
        in_specs=[pl.BlockSpec(memory_space=pltpu.SMEM)],
        out_specs=[
            pl.BlockSpec((1, NPH, 8, BW), lambda h: (h, 0, 0, 0)),
            pl.BlockSpec((1, 8, WX), lambda h: (h, 0, 0)),
        ],
        out_shape=[
            jax.ShapeDtypeStruct((H, NPH, 8, BW), jnp.float32),
            jax.ShapeDtypeStruct((H, 8, WX), jnp.float32),
        ],
    )(rel_bias_table)
    expand = pl.kernel(
        _sc_body,
        out_type=jax.ShapeDtypeStruct((H, N, N), jnp.float32),
        mesh=plsc.VectorSubcoreMesh(
            core_axis_name="c", subcore_axis_name="s",
            num_cores=NC, num_subcores=NS,
        ),
        scratch_types=[
            pltpu.VMEM((8, WX), jnp.float32),
            pltpu.VMEM((3, 8, BW), jnp.float32),
            pltpu.SemaphoreType.DMA((3,)),
            pltpu.SemaphoreType.DMA((3,)),
        ],
    )
    return expand(band, fill)
